# baseline (speedup 1.0000x reference)
import functools

import jax
import jax.numpy as jnp
import numpy as np
from jax.experimental import pallas as pl
from jax.experimental.pallas import tpu as pltpu
from jax.experimental.pallas import tpu_sc as plsc

_NUM_BUCKETS = 32
_N_HEADS = 16
_MAX_DISTANCE = 128
_SEQ = 2048
_VT_W = 4352
_DMA_LAG = 64


_WIN_LO = 1920
_WIN_HI = 2176


def _vtab_body(emb_smem, out_ref):
    win = _WIN_HI - _WIN_LO
    s = jax.lax.broadcasted_iota(jnp.int32, (8, win), 0)
    x = jax.lax.broadcasted_iota(jnp.int32, (8, win), 1) + _WIN_LO
    d = x + s - (_SEQ - 1)
    n = -d
    side = jnp.where(n < 0, _NUM_BUCKETS // 2, 0)
    na = jnp.abs(n)
    max_exact = _NUM_BUCKETS // 4
    is_small = na < max_exact
    nf = jnp.maximum(na, 1).astype(jnp.float32)
    val_large = max_exact + (
        jnp.log(nf / max_exact)
        / np.log(_MAX_DISTANCE / max_exact)
        * (_NUM_BUCKETS // 2 - max_exact)
    ).astype(jnp.int32)
    val_large = jnp.minimum(val_large, _NUM_BUCKETS // 2 - 1)
    bucket = side + jnp.where(is_small, na, val_large)
    for h in range(_N_HEADS):
        v = jnp.full((8, win), emb_smem[0, h], jnp.float32)
        for b in range(1, _NUM_BUCKETS):
            v = jnp.where(bucket == b, emb_smem[b, h], v)
        out_ref[h, :, :_WIN_LO] = jnp.full(
            (8, _WIN_LO), emb_smem[_NUM_BUCKETS // 2 - 1, h], jnp.float32
        )
        out_ref[h, :, _WIN_LO:_WIN_HI] = v
        out_ref[h, :, _WIN_HI:] = jnp.full(
            (8, _VT_W - _WIN_HI), emb_smem[_NUM_BUCKETS - 1, h], jnp.float32
        )


def _build_vtab(embeddings):
    return pl.pallas_call(
        _vtab_body,
        out_shape=jax.ShapeDtypeStruct((_N_HEADS, 8, _VT_W), jnp.float32),
        in_specs=[pl.BlockSpec(memory_space=pltpu.SMEM)],
    )(embeddings)


def _expand_body(vtab_hbm, out_hbm, vt, sem):
    c = jax.lax.axis_index("c")
    s = jax.lax.axis_index("s")
    wid = s * 2 + c
    h = wid // 2
    half = wid % 2
    stage = [
        pltpu.async_copy(
            vtab_hbm.at[h, s8], vt.at[pl.ds(s8 * _VT_W, _VT_W)], sem
        )
        for s8 in range(8)
    ]
    for d in stage:
        d.wait()
    i0 = half * (_SEQ // 2)

    def fire_one(r):
        i = i0 + r
        start = (_SEQ - 1) - i
        s8 = jnp.bitwise_and(start, 7)
        src_off = pl.multiple_of(s8 * _VT_W + (start - s8), 8)
        pltpu.async_copy(
            vt.at[pl.ds(src_off, _SEQ)], out_hbm.at[0, h, i], sem
        )

    def wait_one():
        pltpu.make_async_copy(
            vt.at[pl.ds(0, _SEQ)], out_hbm.at[0, 0, 0], sem
        ).wait()

    def prologue(r, carry):
        fire_one(r)
        return carry

    jax.lax.fori_loop(0, _DMA_LAG, prologue, 0, unroll=False)

    def steady(r, carry):
        fire_one(r)
        wait_one()
        return carry

    jax.lax.fori_loop(_DMA_LAG, _SEQ // 2, steady, 0, unroll=False)

    def drain(r, carry):
        wait_one()
        return carry

    jax.lax.fori_loop(0, _DMA_LAG, drain, 0, unroll=False)


@functools.lru_cache(maxsize=1)
def _make_expand():
    return pl.kernel(
        _expand_body,
        out_type=jax.ShapeDtypeStruct((1, _N_HEADS, _SEQ, _SEQ), jnp.float32),
        mesh=plsc.VectorSubcoreMesh(core_axis_name="c", subcore_axis_name="s"),
        compiler_params=pltpu.CompilerParams(use_tc_tiling_on_sc=False),
        scratch_types=[
            pltpu.VMEM((8 * _VT_W,), jnp.float32),
            pltpu.SemaphoreType.DMA,
        ],
    )


def kernel(q, k, embeddings):
    vtab8 = _build_vtab(embeddings)
    return _make_expand()(vtab8)


# speedup vs baseline: 2.6712x; 2.6712x over previous
import functools

import jax
import jax.numpy as jnp
import numpy as np
from jax.experimental import pallas as pl
from jax.experimental.pallas import tpu as pltpu
from jax.experimental.pallas import tpu_sc as plsc

_NUM_BUCKETS = 32
_N_HEADS = 16
_MAX_DISTANCE = 128
_SEQ = 2048
_PT_W = 4096
_WIN_LO = 1792
_WIN_HI = 2304


def _ptab_body(emb_smem, out_ref):
    h = pl.program_id(0)
    win = _WIN_HI - _WIN_LO
    s = jax.lax.broadcasted_iota(jnp.int32, (8, win), 0)
    x = jax.lax.broadcasted_iota(jnp.int32, (8, win), 1) + _WIN_LO
    max_exact = _NUM_BUCKETS // 4
    e15 = emb_smem[_NUM_BUCKETS // 2 - 1, h]
    e31 = emb_smem[_NUM_BUCKETS - 1, h]
    for p in range(16):
        d = x + (8 * p + 7 - (_SEQ - 1)) - s
        n = -d
        side = jnp.where(n < 0, _NUM_BUCKETS // 2, 0)
        na = jnp.abs(n)
        is_small = na < max_exact
        nf = jnp.maximum(na, 1).astype(jnp.float32)
        val_large = max_exact + (
            jnp.log(nf / max_exact)
            / np.log(_MAX_DISTANCE / max_exact)
            * (_NUM_BUCKETS // 2 - max_exact)
        ).astype(jnp.int32)
        val_large = jnp.minimum(val_large, _NUM_BUCKETS // 2 - 1)
        bucket = side + jnp.where(is_small, na, val_large)
        v = jnp.full((8, win), emb_smem[0, h], jnp.float32)
        for b in range(1, _NUM_BUCKETS):
            v = jnp.where(bucket == b, emb_smem[b, h], v)
        out_ref[0, p, :, :_WIN_LO] = jnp.full((8, _WIN_LO), e15, jnp.float32)
        out_ref[0, p, :, _WIN_LO:_WIN_HI] = v
        out_ref[0, p, :, _WIN_HI:] = jnp.full(
            (8, _PT_W - _WIN_HI), e31, jnp.float32
        )


def _build_ptab(embeddings):
    return pl.pallas_call(
        _ptab_body,
        grid=(_N_HEADS,),
        out_shape=jax.ShapeDtypeStruct((_N_HEADS, 16, 8, _PT_W), jnp.float32),
        in_specs=[pl.BlockSpec(memory_space=pltpu.SMEM)],
        out_specs=pl.BlockSpec((1, 16, 8, _PT_W), lambda hh: (hh, 0, 0, 0)),
    )(embeddings)


def _expand_body(p_hbm, out_hbm, buf0, buf1, sem0, sem1):
    c = jax.lax.axis_index("c")
    s = jax.lax.axis_index("s")
    wid = s * 2 + c
    h = wid // 2
    half = wid % 2
    bufs = (buf0, buf1)
    sems = (sem0, sem1)

    def fire16(buf, sem, l0):
        for t in range(16):
            g = 16 * t + l0
            row0 = pl.multiple_of(8 * g, 8)
            pltpu.async_copy(
                buf.at[:, pl.ds(128 * (15 - t), _SEQ)],
                out_hbm.at[0, h, pl.ds(row0, 8), :],
                sem,
            )

    def wait16(buf, sem):
        for _ in range(16):
            pltpu.make_async_copy(
                buf.at[:, pl.ds(0, _SEQ)],
                out_hbm.at[0, 0, pl.ds(0, 8), :],
                sem,
            ).wait()

    for li in range(8):
        buf, sem = bufs[li % 2], sems[li % 2]
        if li >= 2:
            wait16(buf, sem)
        l0 = half * 8 + li
        p = 15 - l0
        pltpu.sync_copy(p_hbm.at[h, p], buf)
        fire16(buf, sem, l0)
    wait16(bufs[0], sems[0])
    wait16(bufs[1], sems[1])


@functools.lru_cache(maxsize=1)
def _make_expand():
    return pl.kernel(
        _expand_body,
        out_type=jax.ShapeDtypeStruct((1, _N_HEADS, _SEQ, _SEQ), jnp.float32),
        mesh=plsc.VectorSubcoreMesh(core_axis_name="c", subcore_axis_name="s"),
        scratch_types=[
            pltpu.VMEM((8, _PT_W), jnp.float32),
            pltpu.VMEM((8, _PT_W), jnp.float32),
            pltpu.SemaphoreType.DMA,
            pltpu.SemaphoreType.DMA,
        ],
    )


def kernel(q, k, embeddings):
    ptab = _build_ptab(embeddings)
    return _make_expand()(ptab)


# speedup vs baseline: 3.0599x; 1.1455x over previous
import functools

import jax
import jax.numpy as jnp
import numpy as np
from jax.experimental import pallas as pl
from jax.experimental.pallas import tpu as pltpu
from jax.experimental.pallas import tpu_sc as plsc

_NUM_BUCKETS = 32
_N_HEADS = 16
_MAX_DISTANCE = 128
_SEQ = 2048
_PT_W = 4096
_WIN_LO = 1792
_WIN_HI = 2176


def _ptab_body(emb_smem, tmpl_ref, band_ref):
    h = pl.program_id(0)
    win = _WIN_HI - _WIN_LO
    s = jax.lax.broadcasted_iota(jnp.int32, (8, win), 0)
    x = jax.lax.broadcasted_iota(jnp.int32, (8, win), 1) + _WIN_LO
    max_exact = _NUM_BUCKETS // 4
    e15 = emb_smem[_NUM_BUCKETS // 2 - 1, h]
    e31 = emb_smem[_NUM_BUCKETS - 1, h]
    tmpl_ref[0, :, :_WIN_HI] = jnp.full((8, _WIN_HI), e15, jnp.float32)
    tmpl_ref[0, :, _WIN_HI:] = jnp.full((8, _PT_W - _WIN_HI), e31, jnp.float32)
    for p in range(16):
        d = x + (8 * p + 7 - (_SEQ - 1)) - s
        n = -d
        side = jnp.where(n < 0, _NUM_BUCKETS // 2, 0)
        na = jnp.abs(n)
        is_small = na < max_exact
        nf = jnp.maximum(na, 1).astype(jnp.float32)
        val_large = max_exact + (
            jnp.log(nf / max_exact)
            / np.log(_MAX_DISTANCE / max_exact)
            * (_NUM_BUCKETS // 2 - max_exact)
        ).astype(jnp.int32)
        val_large = jnp.minimum(val_large, _NUM_BUCKETS // 2 - 1)
        bucket = side + jnp.where(is_small, na, val_large)
        v = jnp.full((8, win), emb_smem[0, h], jnp.float32)
        for b in range(1, _NUM_BUCKETS):
            v = jnp.where(bucket == b, emb_smem[b, h], v)
        band_ref[0, p] = v


def _build_ptab(embeddings):
    return pl.pallas_call(
        _ptab_body,
        grid=(_N_HEADS,),
        out_shape=(
            jax.ShapeDtypeStruct((_N_HEADS, 8, _PT_W), jnp.float32),
            jax.ShapeDtypeStruct(
                (_N_HEADS, 16, 8, _WIN_HI - _WIN_LO), jnp.float32
            ),
        ),
        in_specs=[pl.BlockSpec(memory_space=pltpu.SMEM)],
        out_specs=(
            pl.BlockSpec((1, 8, _PT_W), lambda hh: (hh, 0, 0)),
            pl.BlockSpec(
                (1, 16, 8, _WIN_HI - _WIN_LO), lambda hh: (hh, 0, 0, 0)
            ),
        ),
    )(embeddings)


def _expand_body(tmpl_hbm, band_hbm, out_hbm, buf0, buf1, sem0, sem1):
    c = jax.lax.axis_index("c")
    s = jax.lax.axis_index("s")
    wid = s * 2 + c
    h = wid // 2
    half = wid % 2
    bufs = (buf0, buf1)
    sems = (sem0, sem1)

    def fire16(buf, sem, l0):
        for t in range(16):
            g = 16 * t + l0
            row0 = pl.multiple_of(8 * g, 8)
            pltpu.async_copy(
                buf.at[:, pl.ds(128 * (15 - t), _SEQ)],
                out_hbm.at[0, h, pl.ds(row0, 8), :],
                sem,
            )

    def wait16(buf, sem):
        for _ in range(16):
            pltpu.make_async_copy(
                buf.at[:, pl.ds(0, _SEQ)],
                out_hbm.at[0, 0, pl.ds(0, 8), :],
                sem,
            ).wait()

    pltpu.sync_copy(tmpl_hbm.at[h], buf0)
    pltpu.sync_copy(tmpl_hbm.at[h], buf1)
    for li in range(8):
        buf, sem = bufs[li % 2], sems[li % 2]
        if li >= 2:
            wait16(buf, sem)
        l0 = half * 8 + li
        p = 15 - l0
        pltpu.sync_copy(
            band_hbm.at[h, p], buf.at[:, pl.ds(_WIN_LO, _WIN_HI - _WIN_LO)]
        )
        fire16(buf, sem, l0)
    wait16(bufs[0], sems[0])
    wait16(bufs[1], sems[1])


@functools.lru_cache(maxsize=1)
def _make_expand():
    return pl.kernel(
        _expand_body,
        out_type=jax.ShapeDtypeStruct((1, _N_HEADS, _SEQ, _SEQ), jnp.float32),
        mesh=plsc.VectorSubcoreMesh(core_axis_name="c", subcore_axis_name="s"),
        scratch_types=[
            pltpu.VMEM((8, _PT_W), jnp.float32),
            pltpu.VMEM((8, _PT_W), jnp.float32),
            pltpu.SemaphoreType.DMA,
            pltpu.SemaphoreType.DMA,
        ],
    )


def kernel(q, k, embeddings):
    tmpl, band = _build_ptab(embeddings)
    return _make_expand()(tmpl, band)


# speedup vs baseline: 3.1856x; 1.0411x over previous
import functools

import jax
import jax.numpy as jnp
import numpy as np
from jax.experimental import pallas as pl
from jax.experimental.pallas import tpu as pltpu
from jax.experimental.pallas import tpu_sc as plsc

_NUM_BUCKETS = 32
_N_HEADS = 16
_MAX_DISTANCE = 128
_SEQ = 2048
_PT_W = 4096
_WIN_LO = 1792
_WIN_HI = 2176


def _ptab_body(emb_smem, tmpl_ref, band_ref):
    h = pl.program_id(0)
    win = _WIN_HI - _WIN_LO
    wide = win + 128
    s = jax.lax.broadcasted_iota(jnp.int32, (8, wide), 0)
    y = jax.lax.broadcasted_iota(jnp.int32, (8, wide), 1) + _WIN_LO
    max_exact = _NUM_BUCKETS // 4
    e15 = emb_smem[_NUM_BUCKETS // 2 - 1, h]
    e31 = emb_smem[_NUM_BUCKETS - 1, h]
    tmpl_ref[0, :, :_WIN_HI] = jnp.full((8, _WIN_HI), e15, jnp.float32)
    tmpl_ref[0, :, _WIN_HI:] = jnp.full((8, _PT_W - _WIN_HI), e31, jnp.float32)
    d = y + (7 - (_SEQ - 1)) - s
    n = -d
    side = jnp.where(n < 0, _NUM_BUCKETS // 2, 0)
    na = jnp.abs(n)
    is_small = na < max_exact
    nf = jnp.maximum(na, 1).astype(jnp.float32)
    val_large = max_exact + (
        jnp.log(nf / max_exact)
        / np.log(_MAX_DISTANCE / max_exact)
        * (_NUM_BUCKETS // 2 - max_exact)
    ).astype(jnp.int32)
    val_large = jnp.minimum(val_large, _NUM_BUCKETS // 2 - 1)
    bucket = side + jnp.where(is_small, na, val_large)
    w = jnp.full((8, wide), emb_smem[0, h], jnp.float32)
    for b in range(1, _NUM_BUCKETS):
        w = jnp.where(bucket == b, emb_smem[b, h], w)
    for p in range(16):
        band_ref[0, p] = w[:, 8 * p : 8 * p + win]


def _build_ptab(embeddings):
    return pl.pallas_call(
        _ptab_body,
        grid=(_N_HEADS,),
        out_shape=(
            jax.ShapeDtypeStruct((_N_HEADS, 8, _PT_W), jnp.float32),
            jax.ShapeDtypeStruct(
                (_N_HEADS, 16, 8, _WIN_HI - _WIN_LO), jnp.float32
            ),
        ),
        in_specs=[pl.BlockSpec(memory_space=pltpu.SMEM)],
        out_specs=(
            pl.BlockSpec((1, 8, _PT_W), lambda hh: (hh, 0, 0)),
            pl.BlockSpec(
                (1, 16, 8, _WIN_HI - _WIN_LO), lambda hh: (hh, 0, 0, 0)
            ),
        ),
    )(embeddings)


def _expand_body(tmpl_hbm, band_hbm, out_hbm, buf0, buf1, sem0, sem1):
    c = jax.lax.axis_index("c")
    s = jax.lax.axis_index("s")
    wid = s * 2 + c
    h = wid // 2
    half = wid % 2
    bufs = (buf0, buf1)
    sems = (sem0, sem1)

    def fire16(buf, sem, l0):
        for t in range(16):
            g = 16 * t + l0
            row0 = pl.multiple_of(8 * g, 8)
            pltpu.async_copy(
                buf.at[:, pl.ds(128 * (15 - t), _SEQ)],
                out_hbm.at[0, h, pl.ds(row0, 8), :],
                sem,
            )

    def wait16(buf, sem):
        for _ in range(16):
            pltpu.make_async_copy(
                buf.at[:, pl.ds(0, _SEQ)],
                out_hbm.at[0, 0, pl.ds(0, 8), :],
                sem,
            ).wait()

    pltpu.sync_copy(tmpl_hbm.at[h], buf0)
    pltpu.sync_copy(tmpl_hbm.at[h], buf1)
    for li in range(8):
        buf, sem = bufs[li % 2], sems[li % 2]
        if li >= 2:
            wait16(buf, sem)
        l0 = half * 8 + li
        p = 15 - l0
        pltpu.sync_copy(
            band_hbm.at[h, p], buf.at[:, pl.ds(_WIN_LO, _WIN_HI - _WIN_LO)]
        )
        fire16(buf, sem, l0)
    wait16(bufs[0], sems[0])
    wait16(bufs[1], sems[1])


@functools.lru_cache(maxsize=1)
def _make_expand():
    return pl.kernel(
        _expand_body,
        out_type=jax.ShapeDtypeStruct((1, _N_HEADS, _SEQ, _SEQ), jnp.float32),
        mesh=plsc.VectorSubcoreMesh(core_axis_name="c", subcore_axis_name="s"),
        scratch_types=[
            pltpu.VMEM((8, _PT_W), jnp.float32),
            pltpu.VMEM((8, _PT_W), jnp.float32),
            pltpu.SemaphoreType.DMA,
            pltpu.SemaphoreType.DMA,
        ],
    )


def kernel(q, k, embeddings):
    tmpl, band = _build_ptab(embeddings)
    return _make_expand()(tmpl, band)


# speedup vs baseline: 3.2857x; 1.0314x over previous
import functools

import jax
import jax.numpy as jnp
import numpy as np
from jax.experimental import pallas as pl
from jax.experimental.pallas import tpu as pltpu
from jax.experimental.pallas import tpu_sc as plsc

_NUM_BUCKETS = 32
_N_HEADS = 16
_MAX_DISTANCE = 128
_SEQ = 2048
_PT_W = 4096
_WIN_LO = 1792
_WIN_HI = 2176


def _ptab_body(emb_smem, tmpl_ref, band_ref):
    win = _WIN_HI - _WIN_LO
    wide = win + 128
    s = jax.lax.broadcasted_iota(jnp.int32, (8, wide), 0)
    y = jax.lax.broadcasted_iota(jnp.int32, (8, wide), 1) + _WIN_LO
    max_exact = _NUM_BUCKETS // 4
    d = y + (7 - (_SEQ - 1)) - s
    n = -d
    side = jnp.where(n < 0, _NUM_BUCKETS // 2, 0)
    na = jnp.abs(n)
    is_small = na < max_exact
    nf = jnp.maximum(na, 1).astype(jnp.float32)
    val_large = max_exact + (
        jnp.log(nf / max_exact)
        / np.log(_MAX_DISTANCE / max_exact)
        * (_NUM_BUCKETS // 2 - max_exact)
    ).astype(jnp.int32)
    val_large = jnp.minimum(val_large, _NUM_BUCKETS // 2 - 1)
    bucket = side + jnp.where(is_small, na, val_large)
    for hh in range(_HEADS_PER_STEP):
        h = pl.program_id(0) * _HEADS_PER_STEP + hh
        e15 = emb_smem[_NUM_BUCKETS // 2 - 1, h]
        e31 = emb_smem[_NUM_BUCKETS - 1, h]
        tmpl_ref[hh, :, :_WIN_HI] = jnp.full((8, _WIN_HI), e15, jnp.float32)
        tmpl_ref[hh, :, _WIN_HI:] = jnp.full(
            (8, _PT_W - _WIN_HI), e31, jnp.float32
        )
        w = jnp.full((8, wide), emb_smem[0, h], jnp.float32)
        for b in range(1, _NUM_BUCKETS):
            w = jnp.where(bucket == b, emb_smem[b, h], w)
        for p in range(16):
            band_ref[hh, p] = w[:, 8 * p : 8 * p + win]


_HEADS_PER_STEP = 4


def _build_ptab(embeddings):
    return pl.pallas_call(
        _ptab_body,
        grid=(_N_HEADS // _HEADS_PER_STEP,),
        out_shape=(
            jax.ShapeDtypeStruct((_N_HEADS, 8, _PT_W), jnp.float32),
            jax.ShapeDtypeStruct(
                (_N_HEADS, 16, 8, _WIN_HI - _WIN_LO), jnp.float32
            ),
        ),
        in_specs=[pl.BlockSpec(memory_space=pltpu.SMEM)],
        out_specs=(
            pl.BlockSpec((_HEADS_PER_STEP, 8, _PT_W), lambda i: (i, 0, 0)),
            pl.BlockSpec(
                (_HEADS_PER_STEP, 16, 8, _WIN_HI - _WIN_LO),
                lambda i: (i, 0, 0, 0),
            ),
        ),
    )(embeddings)


def _expand_body(tmpl_hbm, band_hbm, out_hbm, buf0, buf1, sem0, sem1):
    c = jax.lax.axis_index("c")
    s = jax.lax.axis_index("s")
    wid = s * 2 + c
    h = wid // 2
    half = wid % 2
    bufs = (buf0, buf1)
    sems = (sem0, sem1)

    def fire16(buf, sem, l0):
        for t in range(16):
            g = 16 * t + l0
            row0 = pl.multiple_of(8 * g, 8)
            pltpu.async_copy(
                buf.at[:, pl.ds(128 * (15 - t), _SEQ)],
                out_hbm.at[0, h, pl.ds(row0, 8), :],
                sem,
            )

    def wait16(buf, sem):
        for _ in range(16):
            pltpu.make_async_copy(
                buf.at[:, pl.ds(0, _SEQ)],
                out_hbm.at[0, 0, pl.ds(0, 8), :],
                sem,
            ).wait()

    pltpu.sync_copy(tmpl_hbm.at[h], buf0)
    pltpu.sync_copy(tmpl_hbm.at[h], buf1)
    for li in range(8):
        buf, sem = bufs[li % 2], sems[li % 2]
        if li >= 2:
            wait16(buf, sem)
        l0 = half * 8 + li
        p = 15 - l0
        pltpu.sync_copy(
            band_hbm.at[h, p], buf.at[:, pl.ds(_WIN_LO, _WIN_HI - _WIN_LO)]
        )
        fire16(buf, sem, l0)
    wait16(bufs[0], sems[0])
    wait16(bufs[1], sems[1])


@functools.lru_cache(maxsize=1)
def _make_expand():
    return pl.kernel(
        _expand_body,
        out_type=jax.ShapeDtypeStruct((1, _N_HEADS, _SEQ, _SEQ), jnp.float32),
        mesh=plsc.VectorSubcoreMesh(core_axis_name="c", subcore_axis_name="s"),
        scratch_types=[
            pltpu.VMEM((8, _PT_W), jnp.float32),
            pltpu.VMEM((8, _PT_W), jnp.float32),
            pltpu.SemaphoreType.DMA,
            pltpu.SemaphoreType.DMA,
        ],
    )


def kernel(q, k, embeddings):
    tmpl, band = _build_ptab(embeddings)
    return _make_expand()(tmpl, band)
